# baseline (speedup 1.0000x reference)
import jax
import jax.numpy as jnp
from jax import lax
from jax.experimental import pallas as pl
from jax.experimental.pallas import tpu as pltpu
from jax.experimental.pallas import tpu_sc as plsc

DEPTH = 1000
B_CONST = 1024
DCHUNK = 40
CHUNK_WORDS = DCHUNK * B_CONST
NCHUNKS_PER_SLAB = DEPTH // DCHUNK
L = 16


def _sc_body(xt_hbm, on_hbm, off_hbm, out_hbm, xcol_v, buf_v, on_v, off_v):
    info = plsc.get_sparse_core_info()
    nc = info.num_cores
    nw = nc * info.num_subcores
    s, b = xt_hbm.shape
    n_chunks = s * NCHUNKS_PER_SLAB
    wid = lax.axis_index("s") * nc + lax.axis_index("c")

    pltpu.sync_copy(on_hbm, on_v)
    pltpu.sync_copy(off_hbm, off_v)
    on_vec = on_v[...]
    off_vec = off_v[...]

    def fill(t, _):
        buf_v[pl.ds(t * L, L)] = off_vec
        return 0

    lax.fori_loop(0, CHUNK_WORDS // L, fill, 0)

    lanes = lax.iota(jnp.int32, L)

    def do_chunk(m, _):
        c = wid + nw * m

        @pl.when(c < n_chunks)
        def _():
            j = c // NCHUNKS_PER_SLAB
            dlo = (c % NCHUNKS_PER_SLAB) * DCHUNK
            pltpu.sync_copy(xt_hbm.at[j], xcol_v)

            def poke(k, val):
                v = xcol_v[pl.ds(k * L, L)]
                ld = v - dlo
                mask = (ld >= 0) & (ld < DCHUNK)
                idx = (
                    ((ld >> 3) << 13)
                    + ((ld & 7) << 7)
                    + ((k >> 3) << 10)
                    + ((k & 7) << 4)
                    + lanes
                )
                plsc.store_scatter(buf_v, [idx], val, mask=mask)
                return val

            lax.fori_loop(0, b // L, poke, on_vec)
            pltpu.sync_copy(
                buf_v, out_hbm.at[pl.ds(c * CHUNK_WORDS, CHUNK_WORDS)]
            )
            lax.fori_loop(0, b // L, poke, off_vec)

        return 0

    lax.fori_loop(0, pl.cdiv(n_chunks, nw), do_chunk, 0)


def kernel(x, on_value, off_value):
    B, S = x.shape
    xt = x.T
    on16 = jnp.full((L,), on_value, jnp.float32)
    off16 = jnp.full((L,), off_value, jnp.float32)
    mesh = plsc.VectorSubcoreMesh(core_axis_name="c", subcore_axis_name="s")
    f = pl.kernel(
        _sc_body,
        out_type=jax.ShapeDtypeStruct((S * DEPTH * B, ), jnp.float32),
        mesh=mesh,
        compiler_params=pltpu.CompilerParams(
            use_tc_tiling_on_sc=False, needs_layout_passes=False
        ),
        scratch_types=[
            pltpu.VMEM((B,), jnp.int32),
            pltpu.VMEM((CHUNK_WORDS,), jnp.float32),
            pltpu.VMEM((L,), jnp.float32),
            pltpu.VMEM((L,), jnp.float32),
        ],
    )
    out = f(xt, on16, off16)
    out5 = out.reshape(S, DEPTH // 8, B // 128, 8, 128)
    return out5.transpose(2, 4, 0, 1, 3).reshape(B, S, DEPTH)


# speedup vs baseline: 1.3554x; 1.3554x over previous
import jax
import jax.numpy as jnp
from jax import lax
from jax.experimental import pallas as pl
from jax.experimental.pallas import tpu as pltpu
from jax.experimental.pallas import tpu_sc as plsc

DEPTH = 1000
B_CONST = 1024
DCHUNK = 40
CHUNK_WORDS = DCHUNK * B_CONST
NCHUNKS_PER_SLAB = DEPTH // DCHUNK
L = 16


def _sc_body(
    xt_hbm, on_hbm, off_hbm, out_hbm, xc0, xc1, buf0, buf1, on_v, off_v, s0, s1
):
    info = plsc.get_sparse_core_info()
    nc = info.num_cores
    nw = nc * info.num_subcores
    s, b = xt_hbm.shape
    n_chunks = s * NCHUNKS_PER_SLAB
    wid = lax.axis_index("s") * nc + lax.axis_index("c")
    xcols = (xc0, xc1)
    bufs = (buf0, buf1)
    sems = (s0, s1)
    n_m = pl.cdiv(n_chunks, nw)

    pltpu.sync_copy(on_hbm, on_v)
    pltpu.sync_copy(off_hbm, off_v)
    on_vec = on_v[...]
    off_vec = off_v[...]

    def fill(t, _):
        buf0[pl.ds(t * L, L)] = off_vec
        buf1[pl.ds(t * L, L)] = off_vec
        return 0

    lax.fori_loop(0, CHUNK_WORDS // L, fill, 0)

    lanes = lax.iota(jnp.int32, L)

    def run(t, _):
        for bb in range(2):
            m = 2 * t + bb
            c = wid + nw * m
            xcol = xcols[bb]
            buf = bufs[bb]
            sem = sems[bb]

            @pl.when(c < n_chunks)
            def _():
                @pl.when(t > 0)
                def _wait_restore():
                    pltpu.make_async_copy(
                        buf, out_hbm.at[pl.ds(c * CHUNK_WORDS, CHUNK_WORDS)], sem
                    ).wait()
                    c_prev = c - 2 * nw
                    dlo_prev = (c_prev % NCHUNKS_PER_SLAB) * DCHUNK

                    def restore(k, v_carry):
                        v = xcol[pl.ds(k * L, L)]
                        ld = v - dlo_prev
                        mask = (ld >= 0) & (ld < DCHUNK)
                        idx = (
                            ((ld >> 3) << 13)
                            + ((ld & 7) << 7)
                            + ((k >> 3) << 10)
                            + ((k & 7) << 4)
                            + lanes
                        )
                        plsc.store_scatter(buf, [idx], v_carry, mask=mask)
                        return v_carry

                    lax.fori_loop(0, b // L, restore, off_vec)

                j = c // NCHUNKS_PER_SLAB
                dlo = (c % NCHUNKS_PER_SLAB) * DCHUNK
                pltpu.sync_copy(xt_hbm.at[j], xcol)

                def poke_on(k, v_carry):
                    v = xcol[pl.ds(k * L, L)]
                    ld = v - dlo
                    mask = (ld >= 0) & (ld < DCHUNK)
                    idx = (
                        ((ld >> 3) << 13)
                        + ((ld & 7) << 7)
                        + ((k >> 3) << 10)
                        + ((k & 7) << 4)
                        + lanes
                    )
                    plsc.store_scatter(buf, [idx], v_carry, mask=mask)
                    return v_carry

                lax.fori_loop(0, b // L, poke_on, on_vec)
                pltpu.make_async_copy(
                    buf, out_hbm.at[pl.ds(c * CHUNK_WORDS, CHUNK_WORDS)], sem
                ).start()

        return 0

    lax.fori_loop(0, pl.cdiv(n_m, 2), run, 0)

    for bb in range(2):
        last_c = wid
        pltpu.make_async_copy(
            bufs[bb], out_hbm.at[pl.ds(last_c * 0, CHUNK_WORDS)], sems[bb]
        ).wait()


def kernel(x, on_value, off_value):
    B, S = x.shape
    xt = x.T
    on16 = jnp.full((L,), on_value, jnp.float32)
    off16 = jnp.full((L,), off_value, jnp.float32)
    mesh = plsc.VectorSubcoreMesh(core_axis_name="c", subcore_axis_name="s")
    f = pl.kernel(
        _sc_body,
        out_type=jax.ShapeDtypeStruct((S * DEPTH * B,), jnp.float32),
        mesh=mesh,
        compiler_params=pltpu.CompilerParams(
            use_tc_tiling_on_sc=False, needs_layout_passes=False
        ),
        scratch_types=[
            pltpu.VMEM((B_CONST,), jnp.int32),
            pltpu.VMEM((B_CONST,), jnp.int32),
            pltpu.VMEM((CHUNK_WORDS,), jnp.float32),
            pltpu.VMEM((CHUNK_WORDS,), jnp.float32),
            pltpu.VMEM((L,), jnp.float32),
            pltpu.VMEM((L,), jnp.float32),
            pltpu.SemaphoreType.DMA,
            pltpu.SemaphoreType.DMA,
        ],
    )
    out = f(xt, on16, off16)
    out5 = out.reshape(S, DEPTH // 8, B // 128, 8, 128)
    return out5.transpose(2, 4, 0, 1, 3).reshape(B, S, DEPTH)
